# Initial kernel scaffold; baseline (speedup 1.0000x reference)
import jax
import jax.numpy as jnp
from jax.experimental import pallas as pl


def kernel(input_ids, embedding, head_w, head_b):
    raise NotImplementedError("write your pallas kernel here")


# speedup vs baseline: 1.6526x; 1.6526x over previous
import functools

import jax
import jax.numpy as jnp
from jax import lax
from jax.experimental import pallas as pl
from jax.experimental.pallas import tpu as pltpu
from jax.experimental.pallas import tpu_sc as plsc

_NC = 2
_NS = 16
_NW = _NC * _NS
_LANES = 16


def _scores_tc(e2, w4):
    n_rows = e2.shape[0]
    blk = 2000
    assert n_rows % blk == 0

    def body(e_ref, w_ref, o_ref):
        o_ref[...] = jnp.dot(e_ref[...], w_ref[...],
                             preferred_element_type=jnp.float32,
                             precision=lax.Precision.HIGHEST)

    return pl.pallas_call(
        body,
        grid=(n_rows // blk,),
        in_specs=[
            pl.BlockSpec((blk, 128), lambda i: (i, 0)),
            pl.BlockSpec((128, 4), lambda i: (0, 0)),
        ],
        out_specs=pl.BlockSpec((blk, 4), lambda i: (i, 0)),
        out_shape=jax.ShapeDtypeStruct((n_rows, 4), jnp.float32),
    )(e2, w4)


def _make_sc_pool(B, S, V):
    rows_per_tile = B // _NW
    groups = rows_per_tile // _LANES
    mesh = plsc.VectorSubcoreMesh(core_axis_name="c", subcore_axis_name="s")

    @functools.partial(
        pl.kernel,
        mesh=mesh,
        out_type=jax.ShapeDtypeStruct((B,), jnp.float32),
        scratch_types=[
            pltpu.VMEM((S, rows_per_tile), jnp.int32),
            pltpu.VMEM((S, rows_per_tile), jnp.float32),
            pltpu.VMEM((rows_per_tile,), jnp.float32),
            pltpu.VMEM((_LANES,), jnp.float32),
            pltpu.SemaphoreType.DMA,
        ],
    )
    def sc_pool(scores_hbm, ids_hbm, b_hbm, out_hbm,
                idx_v, vals_v, outb_v, b_v, sem):
        wid = lax.axis_index("s") * _NC + lax.axis_index("c")
        pltpu.sync_copy(ids_hbm.at[wid], idx_v)
        pltpu.sync_copy(b_hbm, b_v)

        def fire(j, carry):
            pltpu.async_copy(scores_hbm.at[idx_v.at[j]], vals_v.at[j], sem)
            return carry

        lax.fori_loop(0, S, fire, 0)

        def drain(j, carry):
            pltpu.make_async_copy(scores_hbm.at[idx_v.at[j]],
                                  vals_v.at[j], sem).wait()
            return carry

        lax.fori_loop(0, S, drain, 0)

        bias = b_v[...]
        for g in range(groups):
            def red(s, acc, _g=g):
                return acc + vals_v[s, pl.ds(_g * _LANES, _LANES)]

            acc = lax.fori_loop(0, S, red, jnp.zeros((_LANES,), jnp.float32))
            outb_v[pl.ds(g * _LANES, _LANES)] = acc + bias

        pltpu.sync_copy(outb_v, out_hbm.at[pl.ds(wid * rows_per_tile,
                                                 rows_per_tile)])

    return sc_pool


def kernel(input_ids, embedding, head_w, head_b):
    B, S = input_ids.shape
    V, D = embedding.shape
    assert D == 32 and V % 4 == 0 and B % (_NW * _LANES) == 0

    e2 = embedding.reshape(V // 4, 4 * D)
    w = head_w[:, 0] * (1.0 / S)
    w4 = (jnp.eye(4, dtype=jnp.float32)[:, None, :] * w[None, :, None]
          ).reshape(4 * D, 4)
    scores = _scores_tc(e2, w4).reshape(V)

    rows_per_tile = B // _NW
    ids_t = input_ids.astype(jnp.int32).reshape(
        _NW, rows_per_tile, S).transpose(0, 2, 1)
    b16 = jnp.broadcast_to(head_b.astype(jnp.float32), (_LANES,))

    return _make_sc_pool(B, S, V)(scores, ids_t, b16)
